# baseline (speedup 1.0000x reference)
import functools

import jax
import jax.numpy as jnp
from jax import lax
from jax.experimental import pallas as pl
from jax.experimental.pallas import tpu as pltpu
from jax.experimental.pallas import tpu_sc as plsc

_B = 16384
_NC = 128
_L = 16
_NW = 32
_ROWS = _B // _NW
_CHUNK = 128
_NCHUNKS = _ROWS // _CHUNK


def _body(x_hbm, y_hbm, w_hbm, b_hbm, emb_hbm, out_hbm,
          idx_v, x_buf, e_buf, w_v, b_v, out_v,
          sem_e0, sem_e1, sem_x0, sem_x1):
    cid = lax.axis_index("c")
    sid = lax.axis_index("s")
    wid = sid * 2 + cid
    base = wid * _ROWS

    pltpu.sync_copy(y_hbm.at[wid], idx_v)
    pltpu.sync_copy(w_hbm.at[0], w_v)
    pltpu.sync_copy(b_hbm, b_v)

    sems_e = [sem_e0, sem_e1]
    sems_x = [sem_x0, sem_x1]

    def start(j):
        s = j % 2
        pltpu.async_copy(emb_hbm.at[idx_v.at[j]], e_buf.at[s], sems_e[s])
        pltpu.async_copy(x_hbm.at[wid, j], x_buf.at[s], sems_x[s])

    def wait(j):
        s = j % 2
        pltpu.make_async_copy(emb_hbm.at[idx_v.at[j]], e_buf.at[s],
                              sems_e[s]).wait()
        pltpu.make_async_copy(x_hbm.at[wid, j], x_buf.at[s],
                              sems_x[s]).wait()

    start(0)

    wg = [w_v[pl.ds(g * _L, _L)] for g in range(_NC // _L)]
    bias_vec = b_v[...]
    last_lane = lax.broadcasted_iota(jnp.int32, (_L,), 0) == (_L - 1)

    for j in range(_NCHUNKS):
        if j + 1 < _NCHUNKS:
            start(j + 1)
        wait(j)
        s = j % 2

        def row(r, carry, s=s, j=j):
            acc = bias_vec
            for g in range(_NC // _L):
                acc += ((e_buf[s, r, pl.ds(g * _L, _L)] + wg[g])
                        * x_buf[s, r, pl.ds(g * _L, _L)])
            tot = plsc.cumsum(acc)
            plsc.store_compressed(out_v.at[pl.ds(j * _CHUNK + r, _L)],
                                  tot, mask=last_lane)
            return carry

        lax.fori_loop(0, _CHUNK, row, 0)

    pltpu.sync_copy(out_v.at[pl.ds(0, _ROWS)], out_hbm.at[pl.ds(base, _ROWS)])


@jax.jit
def kernel(x, y, fc_w, fc_b, emb):
    mesh = plsc.VectorSubcoreMesh(core_axis_name="c", subcore_axis_name="s")
    x4 = x.reshape(_NW, _NCHUNKS, _CHUNK, _NC)
    y3 = y.astype(jnp.int32).reshape(_NW, _NCHUNKS, _CHUNK)
    b16 = jnp.pad(fc_b, (0, _L - 1))
    run = pl.kernel(
        _body,
        out_type=jax.ShapeDtypeStruct((_B,), jnp.float32),
        mesh=mesh,
        compiler_params=pltpu.CompilerParams(needs_layout_passes=False),
        scratch_types=[
            pltpu.VMEM((_NCHUNKS, _CHUNK), jnp.int32),
            pltpu.VMEM((2, _CHUNK, _NC), jnp.float32),
            pltpu.VMEM((2, _CHUNK, _NC), jnp.float32),
            pltpu.VMEM((_NC,), jnp.float32),
            pltpu.VMEM((_L,), jnp.float32),
            pltpu.VMEM((_ROWS + _L,), jnp.float32),
            pltpu.SemaphoreType.DMA,
            pltpu.SemaphoreType.DMA,
            pltpu.SemaphoreType.DMA,
            pltpu.SemaphoreType.DMA,
        ],
    )
    return run(x4, y3, fc_w, b16, emb)
